# Initial kernel scaffold; baseline (speedup 1.0000x reference)
import jax
import jax.numpy as jnp
from jax.experimental import pallas as pl


def kernel(m, rbf, edge_index, W_rbf, W1, b1, W2, b2, W3, b3, W_final):
    raise NotImplementedError("write your pallas kernel here")


# speedup vs baseline: 5.3791x; 5.3791x over previous
import jax
import jax.numpy as jnp
from jax.experimental import pallas as pl
from jax.experimental.pallas import tpu as pltpu

N_NODES = 10000
N_EDGES = 320000
EMB = 128
NUM_RADIAL = 6
NUM_TARGETS = 12

BLOCK_E = 8000
NUM_BLOCKS = N_EDGES // BLOCK_E


def _output_block_kernel(m_ref, rbf_ref, Wr_ref, W1_ref, b1_ref, W2_ref,
                         b2_ref, W3_ref, b3_ref, Wf_ref, out_ref, acc_ref):
    i = pl.program_id(0)

    @pl.when(i == 0)
    def _init():
        acc_ref[...] = jnp.zeros_like(acc_ref)

    w = jax.lax.dot_general(rbf_ref[...], Wr_ref[...],
                            dimension_numbers=(((1,), (1,)), ((), ())),
                            preferred_element_type=jnp.float32,
                            precision=jax.lax.Precision.HIGHEST)
    acc_ref[...] += jnp.sum(m_ref[...] * w, axis=0, keepdims=True)

    @pl.when(i == NUM_BLOCKS - 1)
    def _epilogue():
        n = jnp.float32(N_NODES)
        t = acc_ref[...]
        t = jax.lax.dot_general(t, W1_ref[...], (((1,), (1,)), ((), ())),
                                preferred_element_type=jnp.float32,
                                precision=jax.lax.Precision.HIGHEST) + n * b1_ref[...]
        t = jax.lax.dot_general(t, W2_ref[...], (((1,), (1,)), ((), ())),
                                preferred_element_type=jnp.float32,
                                precision=jax.lax.Precision.HIGHEST) + n * b2_ref[...]
        t = jax.lax.dot_general(t, W3_ref[...], (((1,), (1,)), ((), ())),
                                preferred_element_type=jnp.float32,
                                precision=jax.lax.Precision.HIGHEST) + n * b3_ref[...]
        out_ref[...] = jax.lax.dot_general(t, Wf_ref[...], (((1,), (1,)), ((), ())),
                                           preferred_element_type=jnp.float32,
                                           precision=jax.lax.Precision.HIGHEST)


def kernel(m, rbf, edge_index, W_rbf, W1, b1, W2, b2, W3, b3, W_final):
    del edge_index
    b1r = b1.reshape(1, EMB)
    b2r = b2.reshape(1, EMB)
    b3r = b3.reshape(1, EMB)
    return pl.pallas_call(
        _output_block_kernel,
        grid=(NUM_BLOCKS,),
        in_specs=[
            pl.BlockSpec((BLOCK_E, EMB), lambda i: (i, 0)),
            pl.BlockSpec((BLOCK_E, NUM_RADIAL), lambda i: (i, 0)),
            pl.BlockSpec((EMB, NUM_RADIAL), lambda i: (0, 0)),
            pl.BlockSpec((EMB, EMB), lambda i: (0, 0)),
            pl.BlockSpec((1, EMB), lambda i: (0, 0)),
            pl.BlockSpec((EMB, EMB), lambda i: (0, 0)),
            pl.BlockSpec((1, EMB), lambda i: (0, 0)),
            pl.BlockSpec((EMB, EMB), lambda i: (0, 0)),
            pl.BlockSpec((1, EMB), lambda i: (0, 0)),
            pl.BlockSpec((NUM_TARGETS, EMB), lambda i: (0, 0)),
        ],
        out_specs=pl.BlockSpec((1, NUM_TARGETS), lambda i: (0, 0)),
        out_shape=jax.ShapeDtypeStruct((1, NUM_TARGETS), jnp.float32),
        scratch_shapes=[pltpu.VMEM((1, EMB), jnp.float32)],
    )(m, rbf, W_rbf, W1, b1r, W2, b2r, W3, b3r, W_final)


# speedup vs baseline: 6.8515x; 1.2737x over previous
import jax
import jax.numpy as jnp
from jax.experimental import pallas as pl
from jax.experimental.pallas import tpu as pltpu

N_NODES = 10000
N_EDGES = 320000
EMB = 128
NUM_RADIAL = 6
NUM_TARGETS = 12

BLOCK_E = 8000
NUM_BLOCKS = N_EDGES // BLOCK_E

_ROW = (((1,), (1,)), ((), ()))


def _output_block_kernel(m_ref, rbf_ref, WrT_ref, W1_ref, b1_ref, W2_ref,
                         b2_ref, W3_ref, b3_ref, Wf_ref, out_ref, acc_ref):
    i = pl.program_id(0)

    @pl.when(i == 0)
    def _init():
        acc_ref[...] = jnp.zeros_like(acc_ref)

    acc_ref[...] += jax.lax.dot_general(
        rbf_ref[...], m_ref[...],
        dimension_numbers=(((0,), (0,)), ((), ())),
        preferred_element_type=jnp.float32,
        precision=jax.lax.Precision.DEFAULT)

    @pl.when(i == NUM_BLOCKS - 1)
    def _epilogue():
        n = jnp.float32(N_NODES)
        t = jnp.sum(acc_ref[...] * WrT_ref[...], axis=0, keepdims=True)
        t = jax.lax.dot_general(t, W1_ref[...], _ROW,
                                preferred_element_type=jnp.float32,
                                precision=jax.lax.Precision.HIGHEST) + n * b1_ref[...]
        t = jax.lax.dot_general(t, W2_ref[...], _ROW,
                                preferred_element_type=jnp.float32,
                                precision=jax.lax.Precision.HIGHEST) + n * b2_ref[...]
        t = jax.lax.dot_general(t, W3_ref[...], _ROW,
                                preferred_element_type=jnp.float32,
                                precision=jax.lax.Precision.HIGHEST) + n * b3_ref[...]
        out_ref[...] = jax.lax.dot_general(t, Wf_ref[...], _ROW,
                                           preferred_element_type=jnp.float32,
                                           precision=jax.lax.Precision.HIGHEST)


def kernel(m, rbf, edge_index, W_rbf, W1, b1, W2, b2, W3, b3, W_final):
    del edge_index
    WrT = W_rbf.T
    b1r = b1.reshape(1, EMB)
    b2r = b2.reshape(1, EMB)
    b3r = b3.reshape(1, EMB)
    return pl.pallas_call(
        _output_block_kernel,
        grid=(NUM_BLOCKS,),
        in_specs=[
            pl.BlockSpec((BLOCK_E, EMB), lambda i: (i, 0)),
            pl.BlockSpec((BLOCK_E, NUM_RADIAL), lambda i: (i, 0)),
            pl.BlockSpec((NUM_RADIAL, EMB), lambda i: (0, 0)),
            pl.BlockSpec((EMB, EMB), lambda i: (0, 0)),
            pl.BlockSpec((1, EMB), lambda i: (0, 0)),
            pl.BlockSpec((EMB, EMB), lambda i: (0, 0)),
            pl.BlockSpec((1, EMB), lambda i: (0, 0)),
            pl.BlockSpec((EMB, EMB), lambda i: (0, 0)),
            pl.BlockSpec((1, EMB), lambda i: (0, 0)),
            pl.BlockSpec((NUM_TARGETS, EMB), lambda i: (0, 0)),
        ],
        out_specs=pl.BlockSpec((1, NUM_TARGETS), lambda i: (0, 0)),
        out_shape=jax.ShapeDtypeStruct((1, NUM_TARGETS), jnp.float32),
        scratch_shapes=[pltpu.VMEM((NUM_RADIAL, EMB), jnp.float32)],
    )(m, rbf, WrT, W1, b1r, W2, b2r, W3, b3r, W_final)
